# baseline (speedup 1.0000x reference)
import functools

import jax
import jax.numpy as jnp
from jax import lax
from jax.experimental import pallas as pl
from jax.experimental.pallas import tpu as pltpu
from jax.experimental.pallas import tpu_sc as plsc

_NC = 2
_NS = 16
_LANES = 16
_NW = _NC * _NS


def _pick_chunk(per_w, dim):
    budget = 500_000
    c = 128
    while c >= 16:
        if per_w % c == 0 and (per_w // c) % 2 == 0 and 2 * c * (dim * 4 + 4) <= budget:
            return c
        c -= 16
    raise ValueError("no viable chunk size")


def _build_sc_lookup(b, t, ncb, vocab, dim):
    n_idx = b * t * ncb
    assert n_idx % _NW == 0
    per_w = n_idx // _NW
    chunk = _pick_chunk(per_w, dim)
    assert chunk % _LANES == 0 and _LANES % ncb == 0
    n_chunks = per_w // chunk

    mesh = plsc.VectorSubcoreMesh(core_axis_name="c", subcore_axis_name="s")

    @functools.partial(
        pl.kernel,
        mesh=mesh,
        out_type=jax.ShapeDtypeStruct((b, t, ncb, dim), jnp.float32),
        scratch_types=[
            pltpu.VMEM((per_w,), jnp.int32),
            pltpu.VMEM((chunk, dim), jnp.float32),
            pltpu.VMEM((chunk, dim), jnp.float32),
            pltpu.SemaphoreType.DMA,
            pltpu.SemaphoreType.DMA,
            pltpu.SemaphoreType.DMA,
            pltpu.SemaphoreType.DMA,
        ],
    )
    def lookup(tokens_hbm, table_hbm, out4_hbm, idx_all, rows_a, rows_b,
               gsem_a, gsem_b, ssem_a, ssem_b):
        out_hbm = out4_hbm.reshape(n_idx, dim)
        wid = lax.axis_index("s") * _NC + lax.axis_index("c")
        base = wid * per_w
        off = (lax.iota(jnp.int32, 16) % ncb) * vocab

        pltpu.sync_copy(tokens_hbm.at[pl.ds(base, per_w)], idx_all)

        def add_off(i, carry):
            sl = pl.ds(i * _LANES, _LANES)
            idx_all[sl] = idx_all[sl] + off
            return carry

        lax.fori_loop(0, per_w // _LANES, add_off, 0)

        rows = (rows_a, rows_b)
        gsem = (gsem_a, gsem_b)
        ssem = (ssem_a, ssem_b)

        def fire_gather(g, p):
            pltpu.async_copy(
                table_hbm.at[idx_all.at[pl.ds(g * chunk, chunk)]],
                rows[p], gsem[p])

        def drain_gather(p):
            pltpu.make_async_copy(out_hbm.at[pl.ds(0, chunk)], rows[p],
                                  gsem[p]).wait()

        def fire_scatter(g, p):
            cbase = base + g * chunk
            pltpu.async_copy(rows[p], out_hbm.at[pl.ds(cbase, chunk)], ssem[p])

        def drain_scatter(p):
            pltpu.make_async_copy(rows[p], out_hbm.at[pl.ds(0, chunk)],
                                  ssem[p]).wait()

        fire_gather(0, 0)
        fire_gather(1, 1)
        drain_gather(0)
        fire_scatter(0, 0)

        def body(s, carry):
            ge = 2 * s + 2
            go = ge + 1
            drain_scatter(0)
            fire_gather(ge, 0)
            drain_gather(1)
            fire_scatter(ge - 1, 1)
            drain_scatter(1)
            fire_gather(go, 1)
            drain_gather(0)
            fire_scatter(ge, 0)
            return carry

        lax.fori_loop(0, (n_chunks - 2) // 2, body, 0)

        drain_gather(1)
        fire_scatter(n_chunks - 1, 1)
        drain_scatter(0)
        drain_scatter(1)

    return lookup


def kernel(in_tokens, table):
    b, t, ncb = in_tokens.shape
    n_rows, dim = table.shape
    vocab = n_rows // ncb
    lookup = _build_sc_lookup(b, t, ncb, vocab, dim)
    return lookup(in_tokens.reshape(b * t * ncb), table)


# speedup vs baseline: 1.0036x; 1.0036x over previous
import functools

import jax
import jax.numpy as jnp
from jax import lax
from jax.experimental import pallas as pl
from jax.experimental.pallas import tpu as pltpu
from jax.experimental.pallas import tpu_sc as plsc

_NC = 2
_NS = 16
_LANES = 16
_NW = _NC * _NS


def _pick_chunk(per_w, dim):
    budget = 500_000
    c = 128
    while c >= 16:
        if per_w % c == 0 and (per_w // c) % 2 == 0 and 2 * c * (dim * 4 + 4) <= budget:
            return c
        c -= 16
    raise ValueError("no viable chunk size")


def _build_sc_lookup(b, t, ncb, vocab, dim):
    n_idx = b * t * ncb
    assert n_idx % _NW == 0
    per_w = n_idx // _NW
    chunk = _pick_chunk(per_w, dim)
    assert chunk % _LANES == 0 and _LANES % ncb == 0
    n_chunks = per_w // chunk

    mesh = plsc.VectorSubcoreMesh(core_axis_name="c", subcore_axis_name="s")

    @functools.partial(
        pl.kernel,
        mesh=mesh,
        out_type=jax.ShapeDtypeStruct((b, t, ncb, dim), jnp.float32),
        scratch_types=[
            pltpu.VMEM((chunk,), jnp.int32),
            pltpu.VMEM((chunk,), jnp.int32),
            pltpu.VMEM((chunk, dim), jnp.float32),
            pltpu.VMEM((chunk, dim), jnp.float32),
            pltpu.SemaphoreType.DMA,
            pltpu.SemaphoreType.DMA,
            pltpu.SemaphoreType.DMA,
            pltpu.SemaphoreType.DMA,
        ],
    )
    def lookup(tokens_hbm, table_hbm, out4_hbm, idx_a, idx_b, rows_a, rows_b,
               gsem_a, gsem_b, ssem_a, ssem_b):
        out_hbm = out4_hbm.reshape(n_idx, dim)
        wid = lax.axis_index("s") * _NC + lax.axis_index("c")
        base = wid * per_w
        off = (lax.iota(jnp.int32, 16) % ncb) * vocab

        idx = (idx_a, idx_b)
        rows = (rows_a, rows_b)
        gsem = (gsem_a, gsem_b)
        ssem = (ssem_a, ssem_b)

        def fire_gather(g, p):
            cbase = base + g * chunk
            pltpu.sync_copy(tokens_hbm.at[pl.ds(cbase, chunk)], idx[p])
            for i in range(chunk // _LANES):
                sl = pl.ds(i * _LANES, _LANES)
                idx[p][sl] = idx[p][sl] + off
            pltpu.async_copy(table_hbm.at[idx[p]], rows[p], gsem[p])

        def drain_gather(p):
            pltpu.make_async_copy(out_hbm.at[pl.ds(0, chunk)], rows[p],
                                  gsem[p]).wait()

        def fire_scatter(g, p):
            cbase = base + g * chunk
            pltpu.async_copy(rows[p], out_hbm.at[pl.ds(cbase, chunk)], ssem[p])

        def drain_scatter(p):
            pltpu.make_async_copy(rows[p], out_hbm.at[pl.ds(0, chunk)],
                                  ssem[p]).wait()

        fire_gather(0, 0)
        fire_gather(1, 1)
        drain_gather(0)
        fire_scatter(0, 0)

        def body(s, carry):
            ge = 2 * s + 2
            go = ge + 1
            drain_scatter(0)
            fire_gather(ge, 0)
            drain_gather(1)
            fire_scatter(ge - 1, 1)
            drain_scatter(1)
            fire_gather(go, 1)
            drain_gather(0)
            fire_scatter(ge, 0)
            return carry

        lax.fori_loop(0, (n_chunks - 2) // 2, body, 0)

        drain_gather(1)
        fire_scatter(n_chunks - 1, 1)
        drain_scatter(0)
        drain_scatter(1)

    return lookup


def kernel(in_tokens, table):
    b, t, ncb = in_tokens.shape
    n_rows, dim = table.shape
    vocab = n_rows // ncb
    lookup = _build_sc_lookup(b, t, ncb, vocab, dim)
    return lookup(in_tokens.reshape(b * t * ncb), table)
